# Initial kernel scaffold; baseline (speedup 1.0000x reference)
import jax
import jax.numpy as jnp
from jax.experimental import pallas as pl


def kernel(x, edge_index, batch, emb, W1, b1, W2, b2, W3, b3, Wc1, bc1, Wc2, bc2):
    raise NotImplementedError("write your pallas kernel here")


# speedup vs baseline: 6.6878x; 6.6878x over previous
import functools

import jax
import jax.numpy as jnp
from jax import lax
from jax.experimental import pallas as pl
from jax.experimental.pallas import tpu as pltpu
from jax.experimental.pallas import tpu_sc as plsc

NN = 50000
EE = 800000
GG = 64
DD = 64
HD = 32
NP = 50176
NTS = 16
BLK = 512
NBLK = NP // BLK
ROWS_PT = NP // NTS
CHUNK = 128


def _sc_scatter_body(table_hbm, src_hbm, dst_hbm, zero_hbm, out_hbm,
                     shared, idxs, idxs2, idxd, rows, sem):
    c = lax.axis_index("c")
    s = lax.axis_index("s")
    nchunks = src_hbm.shape[1]

    @pl.when(s == 0)
    def _init():
        pltpu.sync_copy(zero_hbm, shared)

    plsc.subcore_barrier()

    off = c * NP

    def chunk(j, _):
        pltpu.sync_copy(src_hbm.at[s, j], idxs)
        pltpu.sync_copy(dst_hbm.at[s, j], idxd)
        for k in range(CHUNK // 16):
            sl = pl.ds(k * 16, 16)
            idxs2[sl] = idxs[sl] + off
        pltpu.async_copy(table_hbm.at[idxs2], rows, sem).wait()
        pltpu.sync_copy(rows, shared.at[idxd], add=True)
        return _

    lax.fori_loop(0, nchunks, chunk, None)
    plsc.subcore_barrier()
    pltpu.sync_copy(shared.at[pl.ds(s * ROWS_PT, ROWS_PT), :],
                    out_hbm.at[pl.ds(c * NP + s * ROWS_PT, ROWS_PT), :])


def _make_sc_scatter(nchunks):
    mesh = plsc.VectorSubcoreMesh(core_axis_name="c", subcore_axis_name="s")
    return functools.partial(
        pl.kernel, mesh=mesh,
        compiler_params=pltpu.CompilerParams(use_tc_tiling_on_sc=False),
        out_type=jax.ShapeDtypeStruct((2 * NP, HD), jnp.float32),
        scratch_types=[
            pltpu.VMEM_SHARED((NP, HD), jnp.float32),
            pltpu.VMEM((CHUNK,), jnp.int32),
            pltpu.VMEM((CHUNK,), jnp.int32),
            pltpu.VMEM((CHUNK,), jnp.int32),
            pltpu.VMEM((CHUNK, HD), jnp.float32),
            pltpu.SemaphoreType.DMA,
        ],
    )(_sc_scatter_body)


_DOT = dict(preferred_element_type=jnp.float32)
_DOTX = dict(preferred_element_type=jnp.float32,
             precision=lax.Precision.HIGHEST)


def _tc_first_body(h0, h1, deg, w, o0, o1, d8):
    hh = jnp.concatenate([h0[...], h1[...]], axis=1)
    dinv = lax.rsqrt(deg[:, :1] + 1.0)
    mp = dinv * jnp.dot(hh, w[...], **_DOT)
    o0[...] = mp[:, :HD]
    o1[...] = mp[:, HD:]
    d8[...] = jnp.broadcast_to(dinv, (BLK, 8))


def _tc_mid_body(s0, s1, m0, m1, d8, b, w, o0, o1):
    sc = jnp.concatenate([s0[...], s1[...]], axis=1)
    mp = jnp.concatenate([m0[...], m1[...]], axis=1)
    dinv = d8[:, :1]
    h = jnp.maximum(dinv * (sc + mp) + b[...], 0.0)
    mn = dinv * jnp.dot(h, w[...], **_DOT)
    o0[...] = mn[:, :HD]
    o1[...] = mn[:, HD:]


def _tc_pool_body(s0, s1, m0, m1, d8, b, bat, wc1, bc1, wc2, bc2, out,
                  sums, cnts, maxs):
    i = pl.program_id(0)

    @pl.when(i == 0)
    def _init():
        sums[...] = jnp.zeros((GG, DD), jnp.float32)
        cnts[...] = jnp.zeros((GG, DD), jnp.float32)
        maxs[...] = jnp.full((GG, DD), -1e30, jnp.float32)

    sc = jnp.concatenate([s0[...], s1[...]], axis=1)
    mp = jnp.concatenate([m0[...], m1[...]], axis=1)
    dinv = d8[:, :1]
    h = jnp.maximum(dinv * (sc + mp) + b[...], 0.0)

    gids = lax.broadcasted_iota(jnp.int32, (BLK, GG), 1).astype(jnp.float32)
    oh = (bat[...] == gids).astype(jnp.float32)
    dn = (((0,), (0,)), ((), ()))
    sums[...] += lax.dot_general(oh, h, dn, **_DOTX)
    cnts[...] += lax.dot_general(oh, jnp.ones_like(h), dn, **_DOTX)

    def upd(g, _):
        mm = jnp.where(bat[...] == g.astype(jnp.float32), h, -1e30)
        bm = jnp.max(mm, axis=0, keepdims=True)
        maxs[pl.ds(g, 1), :] = jnp.maximum(maxs[pl.ds(g, 1), :], bm)
        return _

    lax.fori_loop(0, GG, upd, None)

    @pl.when(i == NBLK - 1)
    def _fin():
        cnt = cnts[...]
        mean = sums[...] / jnp.maximum(cnt, 1.0)
        mx = jnp.where(cnt > 0.0, maxs[...], 0.0)
        hg = jnp.concatenate([mean, mx], axis=1)
        hid = jnp.maximum(jnp.dot(hg, wc1[...], **_DOT) + bc1[...], 0.0)
        out[...] = jnp.dot(hid, wc2[...], **_DOT) + bc2[...]


def _half_spec():
    return pl.BlockSpec((BLK, HD), lambda i: (i, 0))


def _full_spec(shape):
    return pl.BlockSpec(shape, lambda i: tuple(0 for _ in shape))


def _tc_first(h0, h1, deg, w):
    return pl.pallas_call(
        _tc_first_body,
        grid=(NBLK,),
        in_specs=[_half_spec(), _half_spec(),
                  pl.BlockSpec((BLK, HD), lambda i: (i, 0)),
                  _full_spec((DD, DD))],
        out_specs=[_half_spec(), _half_spec(),
                   pl.BlockSpec((BLK, 8), lambda i: (i, 0))],
        out_shape=[jax.ShapeDtypeStruct((NP, HD), jnp.float32),
                   jax.ShapeDtypeStruct((NP, HD), jnp.float32),
                   jax.ShapeDtypeStruct((NP, 8), jnp.float32)],
    )(h0, h1, deg, w)


def _tc_mid(s0, s1, m0, m1, d8, b, w):
    return pl.pallas_call(
        _tc_mid_body,
        grid=(NBLK,),
        in_specs=[_half_spec(), _half_spec(), _half_spec(), _half_spec(),
                  pl.BlockSpec((BLK, 8), lambda i: (i, 0)),
                  _full_spec((1, DD)), _full_spec((DD, DD))],
        out_specs=[_half_spec(), _half_spec()],
        out_shape=[jax.ShapeDtypeStruct((NP, HD), jnp.float32),
                   jax.ShapeDtypeStruct((NP, HD), jnp.float32)],
    )(s0, s1, m0, m1, d8, b, w)


def _tc_pool(s0, s1, m0, m1, d8, b, bat3, wc1, bc1, wc2p, bc2p):
    return pl.pallas_call(
        _tc_pool_body,
        grid=(NBLK,),
        in_specs=[_half_spec(), _half_spec(), _half_spec(), _half_spec(),
                  pl.BlockSpec((BLK, 8), lambda i: (i, 0)),
                  _full_spec((1, DD)),
                  pl.BlockSpec((BLK, GG), lambda i: (i, 0)),
                  _full_spec((2 * DD, DD)), _full_spec((1, DD)),
                  _full_spec((DD, 128)), _full_spec((1, 128))],
        out_specs=pl.BlockSpec((GG, 128), lambda i: (0, 0)),
        out_shape=jax.ShapeDtypeStruct((GG, 128), jnp.float32),
        scratch_shapes=[pltpu.VMEM((GG, DD), jnp.float32),
                        pltpu.VMEM((GG, DD), jnp.float32),
                        pltpu.VMEM((GG, DD), jnp.float32)],
    )(s0, s1, m0, m1, d8, b, bat3, wc1, bc1, wc2p, bc2p)


def _pad_idx(a, total, fill):
    a = a.astype(jnp.int32)
    a = jnp.concatenate(
        [a, jnp.full((total - a.shape[0],), fill, jnp.int32)])
    return a.reshape(NTS, -1, CHUNK)


def _flat_table(m):
    return jnp.concatenate([m[:, :HD], m[:, HD:]], axis=0)


def kernel(x, edge_index, batch, emb, W1, b1, W2, b2, W3, b3,
           Wc1, bc1, Wc2, bc2):
    f32 = jnp.float32
    tok = x[:, 0].astype(jnp.int32)

    ep_emb = NTS * 25 * 128
    src_emb = _pad_idx(tok, ep_emb, 0)
    dst_emb = _pad_idx(jnp.arange(NN, dtype=jnp.int32), ep_emb, NN)
    ep_edge = NTS * 391 * 128
    src_e = _pad_idx(edge_index[0], ep_edge, 0)
    dst_e = _pad_idx(edge_index[1], ep_edge, NN)

    zero = jnp.zeros((NP, HD), f32)
    scat_small = _make_sc_scatter(25)
    scat_big = _make_sc_scatter(391)

    embp = jnp.zeros((NP, DD), f32).at[:emb.shape[0], :].set(emb)
    hflat = scat_small(_flat_table(embp), src_emb, dst_emb, zero)
    h0, h1 = hflat[:NP], hflat[NP:]

    ones_tab = jnp.ones((2 * NP, HD), f32)
    degflat = scat_big(ones_tab, dst_e, dst_e, zero)
    deg = degflat[:NP]

    m0, m1, d8 = _tc_first(h0, h1, deg, W1)

    b1r = b1.reshape(1, DD)
    b2r = b2.reshape(1, DD)
    b3r = b3.reshape(1, DD)

    sflat = scat_big(jnp.concatenate([m0, m1], axis=0), src_e, dst_e, zero)
    m0, m1 = _tc_mid(sflat[:NP], sflat[NP:], m0, m1, d8, b1r, W2)

    sflat = scat_big(jnp.concatenate([m0, m1], axis=0), src_e, dst_e, zero)
    m0, m1 = _tc_mid(sflat[:NP], sflat[NP:], m0, m1, d8, b2r, W3)

    sflat = scat_big(jnp.concatenate([m0, m1], axis=0), src_e, dst_e, zero)

    batv = jnp.concatenate(
        [batch.astype(f32), jnp.full((NP - NN,), GG, f32)])
    batp = jnp.broadcast_to(batv[:, None], (NP, GG))
    wc2p = jnp.zeros((DD, 128), f32).at[:, :2].set(Wc2)
    bc2p = jnp.zeros((1, 128), f32).at[0, :2].set(bc2)
    out = _tc_pool(sflat[:NP], sflat[NP:], m0, m1, d8, b3r,
                   batp, Wc1, bc1.reshape(1, DD), wc2p, bc2p)
    return out[:, :2]


# speedup vs baseline: 7.4745x; 1.1176x over previous
import functools

import jax
import jax.numpy as jnp
from jax import lax
from jax.experimental import pallas as pl
from jax.experimental.pallas import tpu as pltpu
from jax.experimental.pallas import tpu_sc as plsc

NN = 50000
EE = 800000
GG = 64
DD = 64
HD = 32
NP = 50176
NTS = 16
BLK = 512
NBLK = NP // BLK
ROWS_PT = NP // NTS
CHUNK = 128


def _sc_scatter_body(table_hbm, src_hbm, dst_hbm, zero_hbm, out_hbm,
                     shared, idxs, idxs2, idxd, rows, sem):
    c = lax.axis_index("c")
    s = lax.axis_index("s")
    nchunks = src_hbm.shape[1]

    @pl.when(s == 0)
    def _init():
        pltpu.sync_copy(zero_hbm, shared)

    plsc.subcore_barrier()

    off = c * NP

    def chunk(j, _):
        pltpu.sync_copy(src_hbm.at[s, j], idxs)
        pltpu.sync_copy(dst_hbm.at[s, j], idxd)
        for k in range(CHUNK // 16):
            sl = pl.ds(k * 16, 16)
            idxs2[sl] = idxs[sl] + off
        pltpu.async_copy(table_hbm.at[idxs2], rows, sem).wait()
        pltpu.sync_copy(rows, shared.at[idxd], add=True)
        return _

    lax.fori_loop(0, nchunks, chunk, None)
    plsc.subcore_barrier()
    pltpu.sync_copy(shared.at[pl.ds(s * ROWS_PT, ROWS_PT), :],
                    out_hbm.at[pl.ds(c * NP + s * ROWS_PT, ROWS_PT), :])


def _sc_count_body(ones_hbm, dst_hbm, zero_hbm, out_hbm,
                   shared, idxd, rows, sem):
    c = lax.axis_index("c")
    s = lax.axis_index("s")
    nchunks = dst_hbm.shape[1]

    @pl.when(s == 0)
    def _init():
        pltpu.sync_copy(zero_hbm, shared)

    pltpu.sync_copy(ones_hbm, rows)
    plsc.subcore_barrier()

    def chunk(j, _):
        pltpu.sync_copy(dst_hbm.at[s, j], idxd)
        pltpu.sync_copy(rows, shared.at[idxd], add=True)
        return _

    lax.fori_loop(0, nchunks, chunk, None)
    plsc.subcore_barrier()
    pltpu.sync_copy(shared.at[pl.ds(s * ROWS_PT, ROWS_PT), :],
                    out_hbm.at[pl.ds(c * NP + s * ROWS_PT, ROWS_PT), :])


def _make_sc_count():
    mesh = plsc.VectorSubcoreMesh(core_axis_name="c", subcore_axis_name="s")
    return functools.partial(
        pl.kernel, mesh=mesh,
        compiler_params=pltpu.CompilerParams(use_tc_tiling_on_sc=False),
        out_type=jax.ShapeDtypeStruct((2 * NP, HD), jnp.float32),
        scratch_types=[
            pltpu.VMEM_SHARED((NP, HD), jnp.float32),
            pltpu.VMEM((CHUNK,), jnp.int32),
            pltpu.VMEM((CHUNK, HD), jnp.float32),
            pltpu.SemaphoreType.DMA,
        ],
    )(_sc_count_body)


def _make_sc_scatter(nchunks):
    mesh = plsc.VectorSubcoreMesh(core_axis_name="c", subcore_axis_name="s")
    return functools.partial(
        pl.kernel, mesh=mesh,
        compiler_params=pltpu.CompilerParams(use_tc_tiling_on_sc=False),
        out_type=jax.ShapeDtypeStruct((2 * NP, HD), jnp.float32),
        scratch_types=[
            pltpu.VMEM_SHARED((NP, HD), jnp.float32),
            pltpu.VMEM((CHUNK,), jnp.int32),
            pltpu.VMEM((CHUNK,), jnp.int32),
            pltpu.VMEM((CHUNK,), jnp.int32),
            pltpu.VMEM((CHUNK, HD), jnp.float32),
            pltpu.SemaphoreType.DMA,
        ],
    )(_sc_scatter_body)


_DOT = dict(preferred_element_type=jnp.float32)
_DOTX = dict(preferred_element_type=jnp.float32,
             precision=lax.Precision.HIGHEST)


def _tc_first_body(h0, h1, deg, w, o0, o1, d8):
    hh = jnp.concatenate([h0[...], h1[...]], axis=1)
    dinv = lax.rsqrt(deg[:, :1] + 1.0)
    mp = dinv * jnp.dot(hh, w[...], **_DOT)
    o0[...] = mp[:, :HD]
    o1[...] = mp[:, HD:]
    d8[...] = jnp.broadcast_to(dinv, (BLK, 8))


def _tc_mid_body(s0, s1, m0, m1, d8, b, w, o0, o1):
    sc = jnp.concatenate([s0[...], s1[...]], axis=1)
    mp = jnp.concatenate([m0[...], m1[...]], axis=1)
    dinv = d8[:, :1]
    h = jnp.maximum(dinv * (sc + mp) + b[...], 0.0)
    mn = dinv * jnp.dot(h, w[...], **_DOT)
    o0[...] = mn[:, :HD]
    o1[...] = mn[:, HD:]


def _tc_pool_body(s0, s1, m0, m1, d8, b, bat, wc1, bc1, wc2, bc2, out,
                  sums, cnts, maxs):
    i = pl.program_id(0)

    @pl.when(i == 0)
    def _init():
        sums[...] = jnp.zeros((GG, DD), jnp.float32)
        cnts[...] = jnp.zeros((GG, DD), jnp.float32)
        maxs[...] = jnp.full((GG, DD), -1e30, jnp.float32)

    sc = jnp.concatenate([s0[...], s1[...]], axis=1)
    mp = jnp.concatenate([m0[...], m1[...]], axis=1)
    dinv = d8[:, :1]
    h = jnp.maximum(dinv * (sc + mp) + b[...], 0.0)

    gids = lax.broadcasted_iota(jnp.int32, (BLK, GG), 1).astype(jnp.float32)
    oh = (bat[...] == gids).astype(jnp.float32)
    dn = (((0,), (0,)), ((), ()))
    sums[...] += lax.dot_general(oh, h, dn, **_DOTX)
    cnts[...] += lax.dot_general(oh, jnp.ones_like(h), dn, **_DOTX)

    def upd(g, _):
        mm = jnp.where(bat[...] == g.astype(jnp.float32), h, -1e30)
        bm = jnp.max(mm, axis=0, keepdims=True)
        maxs[pl.ds(g, 1), :] = jnp.maximum(maxs[pl.ds(g, 1), :], bm)
        return _

    lax.fori_loop(0, GG, upd, None)

    @pl.when(i == NBLK - 1)
    def _fin():
        cnt = cnts[...]
        mean = sums[...] / jnp.maximum(cnt, 1.0)
        mx = jnp.where(cnt > 0.0, maxs[...], 0.0)
        hg = jnp.concatenate([mean, mx], axis=1)
        hid = jnp.maximum(jnp.dot(hg, wc1[...], **_DOT) + bc1[...], 0.0)
        out[...] = jnp.dot(hid, wc2[...], **_DOT) + bc2[...]


def _half_spec():
    return pl.BlockSpec((BLK, HD), lambda i: (i, 0))


def _full_spec(shape):
    return pl.BlockSpec(shape, lambda i: tuple(0 for _ in shape))


def _tc_first(h0, h1, deg, w):
    return pl.pallas_call(
        _tc_first_body,
        grid=(NBLK,),
        in_specs=[_half_spec(), _half_spec(),
                  pl.BlockSpec((BLK, HD), lambda i: (i, 0)),
                  _full_spec((DD, DD))],
        out_specs=[_half_spec(), _half_spec(),
                   pl.BlockSpec((BLK, 8), lambda i: (i, 0))],
        out_shape=[jax.ShapeDtypeStruct((NP, HD), jnp.float32),
                   jax.ShapeDtypeStruct((NP, HD), jnp.float32),
                   jax.ShapeDtypeStruct((NP, 8), jnp.float32)],
    )(h0, h1, deg, w)


def _tc_mid(s0, s1, m0, m1, d8, b, w):
    return pl.pallas_call(
        _tc_mid_body,
        grid=(NBLK,),
        in_specs=[_half_spec(), _half_spec(), _half_spec(), _half_spec(),
                  pl.BlockSpec((BLK, 8), lambda i: (i, 0)),
                  _full_spec((1, DD)), _full_spec((DD, DD))],
        out_specs=[_half_spec(), _half_spec()],
        out_shape=[jax.ShapeDtypeStruct((NP, HD), jnp.float32),
                   jax.ShapeDtypeStruct((NP, HD), jnp.float32)],
    )(s0, s1, m0, m1, d8, b, w)


def _tc_pool(s0, s1, m0, m1, d8, b, bat3, wc1, bc1, wc2p, bc2p):
    return pl.pallas_call(
        _tc_pool_body,
        grid=(NBLK,),
        in_specs=[_half_spec(), _half_spec(), _half_spec(), _half_spec(),
                  pl.BlockSpec((BLK, 8), lambda i: (i, 0)),
                  _full_spec((1, DD)),
                  pl.BlockSpec((BLK, GG), lambda i: (i, 0)),
                  _full_spec((2 * DD, DD)), _full_spec((1, DD)),
                  _full_spec((DD, 128)), _full_spec((1, 128))],
        out_specs=pl.BlockSpec((GG, 128), lambda i: (0, 0)),
        out_shape=jax.ShapeDtypeStruct((GG, 128), jnp.float32),
        scratch_shapes=[pltpu.VMEM((GG, DD), jnp.float32),
                        pltpu.VMEM((GG, DD), jnp.float32),
                        pltpu.VMEM((GG, DD), jnp.float32)],
    )(s0, s1, m0, m1, d8, b, bat3, wc1, bc1, wc2p, bc2p)


def _pad_idx(a, total, fill):
    a = a.astype(jnp.int32)
    a = jnp.concatenate(
        [a, jnp.full((total - a.shape[0],), fill, jnp.int32)])
    return a.reshape(NTS, -1, CHUNK)


def _flat_table(m):
    return jnp.concatenate([m[:, :HD], m[:, HD:]], axis=0)


def kernel(x, edge_index, batch, emb, W1, b1, W2, b2, W3, b3,
           Wc1, bc1, Wc2, bc2):
    f32 = jnp.float32
    tok = x[:, 0].astype(jnp.int32)

    ep_emb = NTS * 25 * 128
    src_emb = _pad_idx(tok, ep_emb, 0)
    dst_emb = _pad_idx(jnp.arange(NN, dtype=jnp.int32), ep_emb, NN)
    ep_edge = NTS * 391 * 128
    src_e = _pad_idx(edge_index[0], ep_edge, 0)
    dst_e = _pad_idx(edge_index[1], ep_edge, NN)

    zero = jnp.zeros((NP, HD), f32)
    scat_small = _make_sc_scatter(25)
    scat_big = _make_sc_scatter(391)

    embp = jnp.zeros((NP, DD), f32).at[:emb.shape[0], :].set(emb)
    hflat = scat_small(_flat_table(embp), src_emb, dst_emb, zero)
    h0, h1 = hflat[:NP], hflat[NP:]

    degflat = _make_sc_count()(jnp.ones((CHUNK, HD), f32), dst_e, zero)
    deg = degflat[:NP]

    m0, m1, d8 = _tc_first(h0, h1, deg, W1)

    b1r = b1.reshape(1, DD)
    b2r = b2.reshape(1, DD)
    b3r = b3.reshape(1, DD)

    sflat = scat_big(jnp.concatenate([m0, m1], axis=0), src_e, dst_e, zero)
    m0, m1 = _tc_mid(sflat[:NP], sflat[NP:], m0, m1, d8, b1r, W2)

    sflat = scat_big(jnp.concatenate([m0, m1], axis=0), src_e, dst_e, zero)
    m0, m1 = _tc_mid(sflat[:NP], sflat[NP:], m0, m1, d8, b2r, W3)

    sflat = scat_big(jnp.concatenate([m0, m1], axis=0), src_e, dst_e, zero)

    batv = jnp.concatenate(
        [batch.astype(f32), jnp.full((NP - NN,), GG, f32)])
    batp = jnp.broadcast_to(batv[:, None], (NP, GG))
    wc2p = jnp.zeros((DD, 128), f32).at[:, :2].set(Wc2)
    bc2p = jnp.zeros((1, 128), f32).at[0, :2].set(bc2)
    out = _tc_pool(sflat[:NP], sflat[NP:], m0, m1, d8, b3r,
                   batp, Wc1, bc1.reshape(1, DD), wc2p, bc2p)
    return out[:, :2]
